# Initial kernel scaffold; baseline (speedup 1.0000x reference)
import functools

import jax
import jax.numpy as jnp
from jax import lax
from jax.experimental import pallas as pl
from jax.experimental.pallas import tpu as pltpu
from jax.experimental.pallas import tpu_sc as plsc

_NUM_SC = 2
_NUM_TILES = 16
_CHUNK = 80
_ZROWS = 125


@functools.lru_cache(maxsize=None)
def _make_seg_sum(n_nodes, dim, n_edges):
    nw = _NUM_SC * _NUM_TILES
    per_tile = n_edges // nw
    assert per_tile * nw == n_edges and per_tile % _CHUNK == 0
    n_chunks = per_tile // _CHUNK
    rows_per_tile = n_nodes // _NUM_TILES
    assert rows_per_tile * _NUM_TILES == n_nodes and rows_per_tile % _ZROWS == 0
    zcopies = rows_per_tile // _ZROWS
    lanes = dim // 16

    mesh = plsc.VectorSubcoreMesh(core_axis_name="c", subcore_axis_name="s")

    @functools.partial(
        pl.kernel,
        out_type=jax.ShapeDtypeStruct((_NUM_SC * n_nodes, dim), jnp.float32),
        mesh=mesh,
        scratch_types=[
            pltpu.VMEM((_CHUNK,), jnp.int32),
            pltpu.VMEM((_CHUNK,), jnp.int32),
            pltpu.VMEM((_CHUNK, dim), jnp.float32),
            pltpu.VMEM((_ZROWS, dim), jnp.float32),
            pltpu.VMEM_SHARED((n_nodes, dim), jnp.float32),
            pltpu.SemaphoreType.DMA,
        ],
    )
    def seg_sum(x_hbm, src_hbm, dst_hbm, out_hbm, sidx, didx, rows, zbuf, acc, sem):
        c = lax.axis_index("c")
        s = lax.axis_index("s")
        wid = c * _NUM_TILES + s

        def zrow(r, carry):
            for u in range(lanes):
                zbuf[r, pl.ds(16 * u, 16)] = jnp.zeros((16,), jnp.float32)
            return carry

        lax.fori_loop(0, _ZROWS, zrow, 0)
        for z in range(zcopies):
            pltpu.sync_copy(
                zbuf, acc.at[pl.ds(s * rows_per_tile + z * _ZROWS, _ZROWS)])
        plsc.subcore_barrier()

        base0 = wid * per_tile

        def body(i, carry):
            base = base0 + i * _CHUNK
            pltpu.sync_copy(src_hbm.at[pl.ds(base, _CHUNK)], sidx)
            pltpu.sync_copy(dst_hbm.at[pl.ds(base, _CHUNK)], didx)
            pltpu.async_copy(x_hbm.at[sidx], rows, sem).wait()
            pltpu.sync_copy(rows, acc.at[didx], add=True)
            return carry

        lax.fori_loop(0, n_chunks, body, 0)
        plsc.subcore_barrier()
        pltpu.sync_copy(
            acc.at[pl.ds(s * rows_per_tile, rows_per_tile)],
            out_hbm.at[pl.ds(c * n_nodes + s * rows_per_tile, rows_per_tile)])

    return seg_sum


def _gin_dense(x_ref, p_ref, w1_ref, b1_ref, g_ref, bt_ref, w2_ref, b2_ref):
    n = x_ref.shape[0]
    h = x_ref[...] + p_ref[:n, :] + p_ref[n:, :]
    t = jnp.dot(h, w1_ref[...], preferred_element_type=jnp.float32) + b1_ref[...]
    m = jnp.mean(t, axis=0, keepdims=True)
    d = t - m
    v = jnp.mean(d * d, axis=0, keepdims=True)
    t = g_ref[...] * d * lax.rsqrt(v + 1e-5) + bt_ref[...]
    t = jnp.maximum(t, 0.0)
    t = jnp.dot(t, w2_ref[...], preferred_element_type=jnp.float32) + b2_ref[...]
    return jnp.maximum(t, 0.0)


def _dense_body(x_ref, p_ref, w1_ref, b1_ref, g_ref, bt_ref, w2_ref, b2_ref,
                o_ref):
    o_ref[...] = _gin_dense(x_ref, p_ref, w1_ref, b1_ref, g_ref, bt_ref,
                            w2_ref, b2_ref)


def _final_body(x_ref, p_ref, batch_ref, w1_ref, b1_ref, g_ref, bt_ref,
                w2_ref, b2_ref, mw1_ref, mb1_ref, mw2_ref, mb2_ref, o_ref):
    h = _gin_dense(x_ref, p_ref, w1_ref, b1_ref, g_ref, bt_ref, w2_ref, b2_ref)
    n = x_ref.shape[0]
    n_graphs = o_ref.shape[0]
    onehot_t = (lax.broadcasted_iota(jnp.int32, (n_graphs, n), 0)
                == batch_ref[...]).astype(jnp.float32)
    pooled = jnp.dot(onehot_t, h, preferred_element_type=jnp.float32)
    u = jnp.maximum(
        jnp.dot(pooled, mw1_ref[...], preferred_element_type=jnp.float32)
        + mb1_ref[...], 0.0)
    o_ref[...] = (jnp.dot(u, mw2_ref[...], preferred_element_type=jnp.float32)
                  + mb2_ref[...])


def kernel(x, edge_index, batch, batch_size, c0_W1, c0_b1, c0_g, c0_bt, c0_W2,
           c0_b2, c1_W1, c1_b1, c1_g, c1_bt, c1_W2, c1_b2, m_W1, m_b1, m_W2,
           m_b2):
    n, d = x.shape
    e = edge_index.shape[1]
    hid = c0_W1.shape[1]
    out_d = m_W2.shape[1]
    n_graphs = batch_size if isinstance(batch_size, int) else 64

    src = edge_index[0]
    dst = edge_index[1]
    row = lambda v: v.reshape(1, -1)

    seg_sum = _make_seg_sum(n, d, e)

    parts0 = seg_sum(x, src, dst)
    h0 = pl.pallas_call(
        _dense_body,
        out_shape=jax.ShapeDtypeStruct((n, hid), jnp.float32),
    )(x, parts0, c0_W1, row(c0_b1), row(c0_g), row(c0_bt), c0_W2, row(c0_b2))

    parts1 = seg_sum(h0, src, dst)
    out = pl.pallas_call(
        _final_body,
        out_shape=jax.ShapeDtypeStruct((n_graphs, out_d), jnp.float32),
    )(h0, parts1, row(batch), c1_W1, row(c1_b1), row(c1_g), row(c1_bt), c1_W2,
      row(c1_b2), m_W1, row(m_b1), m_W2, row(m_b2))
    return out


# speedup vs baseline: 4.8415x; 4.8415x over previous
import functools

import jax
import jax.numpy as jnp
from jax import lax
from jax.experimental import pallas as pl
from jax.experimental.pallas import tpu as pltpu
from jax.experimental.pallas import tpu_sc as plsc

_NUM_SC = 2
_NUM_TILES = 16
_CHUNK = 80
_ZROWS = 128


def _pad_rows(n_nodes):
    per_tile = -(-n_nodes // _NUM_TILES)
    per_tile = -(-per_tile // _ZROWS) * _ZROWS
    return per_tile


@functools.lru_cache(maxsize=None)
def _make_seg_sum(n_nodes, dim, n_edges):
    nw = _NUM_SC * _NUM_TILES
    per_tile = n_edges // nw
    assert per_tile * nw == n_edges and per_tile % _CHUNK == 0
    n_chunks = per_tile // _CHUNK
    rows_per_tile = _pad_rows(n_nodes)
    n_pad = rows_per_tile * _NUM_TILES
    zcopies = rows_per_tile // _ZROWS
    lanes = dim // 16

    mesh = plsc.VectorSubcoreMesh(core_axis_name="c", subcore_axis_name="s")

    @functools.partial(
        pl.kernel,
        out_type=jax.ShapeDtypeStruct((_NUM_SC * n_pad, dim), jnp.float32),
        mesh=mesh,
        scratch_types=[
            pltpu.VMEM((_CHUNK,), jnp.int32),
            pltpu.VMEM((_CHUNK,), jnp.int32),
            pltpu.VMEM((_CHUNK, dim), jnp.float32),
            pltpu.VMEM((_ZROWS, dim), jnp.float32),
            pltpu.VMEM_SHARED((n_pad, dim), jnp.float32),
            pltpu.SemaphoreType.DMA,
        ],
    )
    def seg_sum(x_hbm, src_hbm, dst_hbm, out_hbm, sidx, didx, rows, zbuf, acc, sem):
        c = lax.axis_index("c")
        s = lax.axis_index("s")
        wid = c * _NUM_TILES + s

        def zrow(r, carry):
            for u in range(lanes):
                zbuf[r, pl.ds(16 * u, 16)] = jnp.zeros((16,), jnp.float32)
            return carry

        lax.fori_loop(0, _ZROWS, zrow, 0)
        for z in range(zcopies):
            pltpu.sync_copy(
                zbuf, acc.at[pl.ds(s * rows_per_tile + z * _ZROWS, _ZROWS)])
        plsc.subcore_barrier()

        base0 = wid * per_tile

        def body(i, carry):
            base = base0 + i * _CHUNK
            pltpu.sync_copy(src_hbm.at[pl.ds(base, _CHUNK)], sidx)
            pltpu.sync_copy(dst_hbm.at[pl.ds(base, _CHUNK)], didx)
            pltpu.async_copy(x_hbm.at[sidx], rows, sem).wait()
            pltpu.sync_copy(rows, acc.at[didx], add=True)
            return carry

        lax.fori_loop(0, n_chunks, body, 0)
        plsc.subcore_barrier()
        pltpu.sync_copy(
            acc.at[pl.ds(s * rows_per_tile, rows_per_tile)],
            out_hbm.at[pl.ds(c * n_pad + s * rows_per_tile, rows_per_tile)])

    return seg_sum


def _gin_dense(x_ref, p_ref, w1_ref, b1_ref, g_ref, bt_ref, w2_ref, b2_ref):
    n = x_ref.shape[0]
    n_pad = p_ref.shape[0] // 2
    h = x_ref[...] + p_ref[:n, :] + p_ref[n_pad:n_pad + n, :]
    t = jnp.dot(h, w1_ref[...], preferred_element_type=jnp.float32) + b1_ref[...]
    m = jnp.mean(t, axis=0, keepdims=True)
    d = t - m
    v = jnp.mean(d * d, axis=0, keepdims=True)
    t = g_ref[...] * d * lax.rsqrt(v + 1e-5) + bt_ref[...]
    t = jnp.maximum(t, 0.0)
    t = jnp.dot(t, w2_ref[...], preferred_element_type=jnp.float32) + b2_ref[...]
    return jnp.maximum(t, 0.0)


def _dense_body(x_ref, p_ref, w1_ref, b1_ref, g_ref, bt_ref, w2_ref, b2_ref,
                o_ref):
    o_ref[...] = _gin_dense(x_ref, p_ref, w1_ref, b1_ref, g_ref, bt_ref,
                            w2_ref, b2_ref)


def _final_body(x_ref, p_ref, batch_ref, w1_ref, b1_ref, g_ref, bt_ref,
                w2_ref, b2_ref, mw1_ref, mb1_ref, mw2_ref, mb2_ref, o_ref):
    h = _gin_dense(x_ref, p_ref, w1_ref, b1_ref, g_ref, bt_ref, w2_ref, b2_ref)
    n = x_ref.shape[0]
    n_graphs = o_ref.shape[0]
    onehot_t = (lax.broadcasted_iota(jnp.int32, (n_graphs, n), 0)
                == batch_ref[...]).astype(jnp.float32)
    pooled = jnp.dot(onehot_t, h, preferred_element_type=jnp.float32)
    u = jnp.maximum(
        jnp.dot(pooled, mw1_ref[...], preferred_element_type=jnp.float32)
        + mb1_ref[...], 0.0)
    o_ref[...] = (jnp.dot(u, mw2_ref[...], preferred_element_type=jnp.float32)
                  + mb2_ref[...])


def kernel(x, edge_index, batch, batch_size, c0_W1, c0_b1, c0_g, c0_bt, c0_W2,
           c0_b2, c1_W1, c1_b1, c1_g, c1_bt, c1_W2, c1_b2, m_W1, m_b1, m_W2,
           m_b2):
    n, d = x.shape
    e = edge_index.shape[1]
    hid = c0_W1.shape[1]
    out_d = m_W2.shape[1]
    n_graphs = batch_size if isinstance(batch_size, int) else 64

    src = edge_index[0]
    dst = edge_index[1]
    row = lambda v: v.reshape(1, -1)

    seg_sum = _make_seg_sum(n, d, e)

    parts0 = seg_sum(x, src, dst)
    h0 = pl.pallas_call(
        _dense_body,
        out_shape=jax.ShapeDtypeStruct((n, hid), jnp.float32),
    )(x, parts0, c0_W1, row(c0_b1), row(c0_g), row(c0_bt), c0_W2, row(c0_b2))

    parts1 = seg_sum(h0, src, dst)
    out = pl.pallas_call(
        _final_body,
        out_shape=jax.ShapeDtypeStruct((n_graphs, out_d), jnp.float32),
    )(h0, parts1, row(batch), c1_W1, row(c1_b1), row(c1_g), row(c1_bt), c1_W2,
      row(c1_b2), m_W1, row(m_b1), m_W2, row(m_b2))
    return out


# speedup vs baseline: 11.3619x; 2.3468x over previous
import functools

import jax
import jax.numpy as jnp
from jax import lax
from jax.experimental import pallas as pl
from jax.experimental.pallas import tpu as pltpu
from jax.experimental.pallas import tpu_sc as plsc

_NUM_SC = 2
_NUM_TILES = 16
_CHUNK = 80
_ZROWS = 128


def _pad_rows(n_nodes):
    per_tile = -(-n_nodes // _NUM_TILES)
    per_tile = -(-per_tile // _ZROWS) * _ZROWS
    return per_tile


@functools.lru_cache(maxsize=None)
def _make_seg_sum(n_nodes, dim, n_edges):
    nw = _NUM_SC * _NUM_TILES
    per_tile = n_edges // nw
    assert per_tile * nw == n_edges and per_tile % _CHUNK == 0
    n_chunks = per_tile // _CHUNK
    rows_per_tile = _pad_rows(n_nodes)
    n_pad = rows_per_tile * _NUM_TILES
    zcopies = rows_per_tile // _ZROWS
    lanes = dim // 16

    mesh = plsc.VectorSubcoreMesh(core_axis_name="c", subcore_axis_name="s")

    assert n_chunks % 2 == 1 and n_chunks >= 3

    @functools.partial(
        pl.kernel,
        out_type=jax.ShapeDtypeStruct((_NUM_SC * n_pad, dim), jnp.float32),
        mesh=mesh,
        scratch_types=[
            pltpu.VMEM((per_tile,), jnp.int32),
            pltpu.VMEM((_CHUNK,), jnp.int32),
            pltpu.VMEM((_CHUNK,), jnp.int32),
            pltpu.VMEM((_CHUNK, dim), jnp.float32),
            pltpu.VMEM((_CHUNK, dim), jnp.float32),
            pltpu.VMEM((_ZROWS, dim), jnp.float32),
            pltpu.VMEM_SHARED((n_pad, dim), jnp.float32),
            pltpu.SemaphoreType.DMA,
            pltpu.SemaphoreType.DMA,
            pltpu.SemaphoreType.DMA,
            pltpu.SemaphoreType.DMA,
        ],
    )
    def seg_sum(x_hbm, src_hbm, dst_hbm, out_hbm, sidx, didx0, didx1,
                rows0, rows1, zbuf, acc, sem0, sem1, semd0, semd1):
        c = lax.axis_index("c")
        s = lax.axis_index("s")
        wid = c * _NUM_TILES + s

        pltpu.sync_copy(src_hbm.at[pl.ds(wid * per_tile, per_tile)], sidx)

        def zrow(r, carry):
            for u in range(lanes):
                zbuf[r, pl.ds(16 * u, 16)] = jnp.zeros((16,), jnp.float32)
            return carry

        lax.fori_loop(0, _ZROWS, zrow, 0)
        for z in range(zcopies):
            pltpu.sync_copy(
                zbuf, acc.at[pl.ds(s * rows_per_tile + z * _ZROWS, _ZROWS)])
        plsc.subcore_barrier()

        def start_gather(i, rows, sem):
            return pltpu.async_copy(
                x_hbm.at[sidx.at[pl.ds(i * _CHUNK, _CHUNK)]], rows, sem)

        def wait_gather(i, rows, sem):
            pltpu.make_async_copy(
                x_hbm.at[sidx.at[pl.ds(i * _CHUNK, _CHUNK)]], rows, sem).wait()

        base0 = wid * per_tile

        def start_didx(i, dbuf, semd):
            return pltpu.async_copy(
                dst_hbm.at[pl.ds(base0 + i * _CHUNK, _CHUNK)], dbuf, semd)

        def wait_didx(i, dbuf, semd):
            pltpu.make_async_copy(
                dst_hbm.at[pl.ds(base0 + i * _CHUNK, _CHUNK)], dbuf,
                semd).wait()

        start_didx(0, didx0, semd0)
        start_gather(0, rows0, sem0)

        def pair(p, carry):
            i0 = 2 * p
            start_didx(i0 + 1, didx1, semd1)
            start_gather(i0 + 1, rows1, sem1)
            wait_gather(i0, rows0, sem0)
            wait_didx(i0, didx0, semd0)
            pltpu.sync_copy(rows0, acc.at[didx0], add=True)
            start_didx(i0 + 2, didx0, semd0)
            start_gather(i0 + 2, rows0, sem0)
            wait_gather(i0 + 1, rows1, sem1)
            wait_didx(i0 + 1, didx1, semd1)
            pltpu.sync_copy(rows1, acc.at[didx1], add=True)
            return carry

        lax.fori_loop(0, (n_chunks - 1) // 2, pair, 0)
        wait_gather(n_chunks - 1, rows0, sem0)
        wait_didx(n_chunks - 1, didx0, semd0)
        pltpu.sync_copy(rows0, acc.at[didx0], add=True)

        plsc.subcore_barrier()
        pltpu.sync_copy(
            acc.at[pl.ds(s * rows_per_tile, rows_per_tile)],
            out_hbm.at[pl.ds(c * n_pad + s * rows_per_tile, rows_per_tile)])

    return seg_sum


def _gin_dense(x_ref, p_ref, w1_ref, b1_ref, g_ref, bt_ref, w2_ref, b2_ref):
    n = x_ref.shape[0]
    n_pad = p_ref.shape[0] // 2
    h = x_ref[...] + p_ref[:n, :] + p_ref[n_pad:n_pad + n, :]
    t = jnp.dot(h, w1_ref[...], preferred_element_type=jnp.float32) + b1_ref[...]
    m = jnp.mean(t, axis=0, keepdims=True)
    d = t - m
    v = jnp.mean(d * d, axis=0, keepdims=True)
    t = g_ref[...] * d * lax.rsqrt(v + 1e-5) + bt_ref[...]
    t = jnp.maximum(t, 0.0)
    t = jnp.dot(t, w2_ref[...], preferred_element_type=jnp.float32) + b2_ref[...]
    return jnp.maximum(t, 0.0)


def _dense_body(x_ref, p_ref, w1_ref, b1_ref, g_ref, bt_ref, w2_ref, b2_ref,
                o_ref):
    o_ref[...] = _gin_dense(x_ref, p_ref, w1_ref, b1_ref, g_ref, bt_ref,
                            w2_ref, b2_ref)


def _final_body(x_ref, p_ref, batch_ref, w1_ref, b1_ref, g_ref, bt_ref,
                w2_ref, b2_ref, mw1_ref, mb1_ref, mw2_ref, mb2_ref, o_ref):
    h = _gin_dense(x_ref, p_ref, w1_ref, b1_ref, g_ref, bt_ref, w2_ref, b2_ref)
    n = x_ref.shape[0]
    n_graphs = o_ref.shape[0]
    onehot_t = (lax.broadcasted_iota(jnp.int32, (n_graphs, n), 0)
                == batch_ref[...]).astype(jnp.float32)
    pooled = jnp.dot(onehot_t, h, preferred_element_type=jnp.float32)
    u = jnp.maximum(
        jnp.dot(pooled, mw1_ref[...], preferred_element_type=jnp.float32)
        + mb1_ref[...], 0.0)
    o_ref[...] = (jnp.dot(u, mw2_ref[...], preferred_element_type=jnp.float32)
                  + mb2_ref[...])


def kernel(x, edge_index, batch, batch_size, c0_W1, c0_b1, c0_g, c0_bt, c0_W2,
           c0_b2, c1_W1, c1_b1, c1_g, c1_bt, c1_W2, c1_b2, m_W1, m_b1, m_W2,
           m_b2):
    n, d = x.shape
    e = edge_index.shape[1]
    hid = c0_W1.shape[1]
    out_d = m_W2.shape[1]
    n_graphs = batch_size if isinstance(batch_size, int) else 64

    src = edge_index[0]
    dst = edge_index[1]
    row = lambda v: v.reshape(1, -1)

    seg_sum = _make_seg_sum(n, d, e)

    parts0 = seg_sum(x, src, dst)
    h0 = pl.pallas_call(
        _dense_body,
        out_shape=jax.ShapeDtypeStruct((n, hid), jnp.float32),
    )(x, parts0, c0_W1, row(c0_b1), row(c0_g), row(c0_bt), c0_W2, row(c0_b2))

    parts1 = seg_sum(h0, src, dst)
    out = pl.pallas_call(
        _final_body,
        out_shape=jax.ShapeDtypeStruct((n_graphs, out_d), jnp.float32),
    )(h0, parts1, row(batch), c1_W1, row(c1_b1), row(c1_g), row(c1_bt), c1_W2,
      row(c1_b2), m_W1, row(m_b1), m_W2, row(m_b2))
    return out


# speedup vs baseline: 13.0311x; 1.1469x over previous
import functools

import jax
import jax.numpy as jnp
from jax import lax
from jax.experimental import pallas as pl
from jax.experimental.pallas import tpu as pltpu
from jax.experimental.pallas import tpu_sc as plsc

_NUM_SC = 2
_NUM_TILES = 16
_CHUNK = 80
_ZROWS = 128


def _pad_rows(n_nodes):
    per_tile = -(-n_nodes // _NUM_TILES)
    per_tile = -(-per_tile // _ZROWS) * _ZROWS
    return per_tile


@functools.lru_cache(maxsize=None)
def _make_seg_sum(n_nodes, dim, n_edges):
    nw = _NUM_SC * _NUM_TILES
    per_tile = n_edges // nw
    assert per_tile * nw == n_edges and per_tile % _CHUNK == 0
    n_chunks = per_tile // _CHUNK
    rows_per_tile = _pad_rows(n_nodes)
    n_pad = rows_per_tile * _NUM_TILES
    zcopies = rows_per_tile // _ZROWS
    lanes = dim // 16

    mesh = plsc.VectorSubcoreMesh(core_axis_name="c", subcore_axis_name="s")

    _NBUF = 3
    assert n_chunks >= 8
    _G3 = (n_chunks - 3) // 3

    @functools.partial(
        pl.kernel,
        out_type=jax.ShapeDtypeStruct((_NUM_SC * n_pad, dim), jnp.float32),
        mesh=mesh,
        scratch_types=[
            pltpu.VMEM((per_tile,), jnp.int32),
            [pltpu.VMEM((_CHUNK,), jnp.int32)] * _NBUF,
            [pltpu.VMEM((_CHUNK, dim), jnp.float32)] * _NBUF,
            pltpu.VMEM_SHARED((n_pad, dim), jnp.float32),
            [pltpu.SemaphoreType.DMA] * _NBUF,
            [pltpu.SemaphoreType.DMA] * _NBUF,
            [pltpu.SemaphoreType.DMA] * _NBUF,
        ],
    )
    def seg_sum(x_hbm, src_hbm, dst_hbm, out_hbm, sidx, didxs, rows,
                acc, gsems, dsems, ssems):
        c = lax.axis_index("c")
        s = lax.axis_index("s")
        wid = c * _NUM_TILES + s
        base0 = wid * per_tile

        pltpu.sync_copy(src_hbm.at[pl.ds(base0, per_tile)], sidx)

        def zrow(r, carry):
            for u in range(lanes):
                rows[0][r, pl.ds(16 * u, 16)] = jnp.zeros((16,), jnp.float32)
            return carry

        lax.fori_loop(0, _CHUNK, zrow, 0)
        for z in range(rows_per_tile // _CHUNK):
            pltpu.sync_copy(
                rows[0],
                acc.at[pl.ds(s * rows_per_tile + z * _CHUNK, _CHUNK)])
        plsc.subcore_barrier()

        def start_chunk(i, b):
            pltpu.async_copy(
                dst_hbm.at[pl.ds(base0 + i * _CHUNK, _CHUNK)], didxs[b],
                dsems[b])
            pltpu.async_copy(
                x_hbm.at[sidx.at[pl.ds(i * _CHUNK, _CHUNK)]], rows[b],
                gsems[b])

        def wait_chunk(i, b):
            pltpu.make_async_copy(
                x_hbm.at[sidx.at[pl.ds(i * _CHUNK, _CHUNK)]], rows[b],
                gsems[b]).wait()
            pltpu.make_async_copy(
                dst_hbm.at[pl.ds(base0 + i * _CHUNK, _CHUNK)], didxs[b],
                dsems[b]).wait()

        def wait_scatter(b):
            pltpu.make_async_copy(rows[b], acc.at[didxs[b]], ssems[b]).wait()

        def stage(i, b, prefetch, wait_prev):
            wait_chunk(i, b)
            pltpu.async_copy(rows[b], acc.at[didxs[b]], ssems[b], add=True)
            nb = (b + 2) % _NBUF
            if wait_prev:
                wait_scatter(nb)
            if prefetch:
                start_chunk(i + 2, nb)

        for j in range(2):
            start_chunk(j, j)
        stage(0, 0, True, False)
        for j in range(1, 3):
            stage(j, j, True, True)

        def group(p, carry):
            i0 = 3 * p
            for u in range(3):
                stage(i0 + u, u, True, True)
            return carry

        lax.fori_loop(1, _G3, group, 0)
        for i in range(3 * _G3, n_chunks):
            stage(i, i % _NBUF, i + 2 < n_chunks, True)
        wait_scatter((n_chunks - 1) % _NBUF)

        plsc.subcore_barrier()
        pltpu.sync_copy(
            acc.at[pl.ds(s * rows_per_tile, rows_per_tile)],
            out_hbm.at[pl.ds(c * n_pad + s * rows_per_tile, rows_per_tile)])

    return seg_sum


def _gin_dense(x_ref, p_ref, w1_ref, b1_ref, g_ref, bt_ref, w2_ref, b2_ref):
    n = x_ref.shape[0]
    n_pad = p_ref.shape[0] // 2
    h = x_ref[...] + p_ref[:n, :] + p_ref[n_pad:n_pad + n, :]
    t = jnp.dot(h, w1_ref[...], preferred_element_type=jnp.float32) + b1_ref[...]
    m = jnp.mean(t, axis=0, keepdims=True)
    d = t - m
    v = jnp.mean(d * d, axis=0, keepdims=True)
    t = g_ref[...] * d * lax.rsqrt(v + 1e-5) + bt_ref[...]
    t = jnp.maximum(t, 0.0)
    t = jnp.dot(t, w2_ref[...], preferred_element_type=jnp.float32) + b2_ref[...]
    return jnp.maximum(t, 0.0)


def _dense_body(x_ref, p_ref, w1_ref, b1_ref, g_ref, bt_ref, w2_ref, b2_ref,
                o_ref):
    o_ref[...] = _gin_dense(x_ref, p_ref, w1_ref, b1_ref, g_ref, bt_ref,
                            w2_ref, b2_ref)


def _final_body(x_ref, p_ref, batch_ref, w1_ref, b1_ref, g_ref, bt_ref,
                w2_ref, b2_ref, mw1_ref, mb1_ref, mw2_ref, mb2_ref, o_ref):
    h = _gin_dense(x_ref, p_ref, w1_ref, b1_ref, g_ref, bt_ref, w2_ref, b2_ref)
    n = x_ref.shape[0]
    n_graphs = o_ref.shape[0]
    onehot_t = (lax.broadcasted_iota(jnp.int32, (n_graphs, n), 0)
                == batch_ref[...]).astype(jnp.float32)
    pooled = jnp.dot(onehot_t, h, preferred_element_type=jnp.float32)
    u = jnp.maximum(
        jnp.dot(pooled, mw1_ref[...], preferred_element_type=jnp.float32)
        + mb1_ref[...], 0.0)
    o_ref[...] = (jnp.dot(u, mw2_ref[...], preferred_element_type=jnp.float32)
                  + mb2_ref[...])


def kernel(x, edge_index, batch, batch_size, c0_W1, c0_b1, c0_g, c0_bt, c0_W2,
           c0_b2, c1_W1, c1_b1, c1_g, c1_bt, c1_W2, c1_b2, m_W1, m_b1, m_W2,
           m_b2):
    n, d = x.shape
    e = edge_index.shape[1]
    hid = c0_W1.shape[1]
    out_d = m_W2.shape[1]
    n_graphs = batch_size if isinstance(batch_size, int) else 64

    src = edge_index[0]
    dst = edge_index[1]
    row = lambda v: v.reshape(1, -1)

    seg_sum = _make_seg_sum(n, d, e)

    parts0 = seg_sum(x, src, dst)
    h0 = pl.pallas_call(
        _dense_body,
        out_shape=jax.ShapeDtypeStruct((n, hid), jnp.float32),
    )(x, parts0, c0_W1, row(c0_b1), row(c0_g), row(c0_bt), c0_W2, row(c0_b2))

    parts1 = seg_sum(h0, src, dst)
    out = pl.pallas_call(
        _final_body,
        out_shape=jax.ShapeDtypeStruct((n_graphs, out_d), jnp.float32),
    )(h0, parts1, row(batch), c1_W1, row(c1_b1), row(c1_g), row(c1_bt), c1_W2,
      row(c1_b2), m_W1, row(m_b1), m_W2, row(m_b2))
    return out
